# Initial kernel scaffold; baseline (speedup 1.0000x reference)
import jax
import jax.numpy as jnp
from jax.experimental import pallas as pl


def kernel(x, edge_index, edge_attr, u, batch, W1, b1, W2, b2, W3, b3):
    raise NotImplementedError("write your pallas kernel here")


# speedup vs baseline: 46.0213x; 46.0213x over previous
import functools

import jax
import jax.numpy as jnp
from jax import lax
from jax.experimental import pallas as pl
from jax.experimental.pallas import tpu as pltpu
from jax.experimental.pallas import tpu_sc as plsc

N_NODES = 100000
N_PAD = 100096
N_EDGES = 3200000
D = 8
NC = 2
NS = 16
NW = NC * NS
CH = 2048
ITERS = 49
E_PER_W = CH * ITERS
E_PAD = NW * E_PER_W
ROWS_PER_TILE = N_PAD // NS


def _sc_agg(x_pad, row, col, zeros):
    mesh = plsc.VectorSubcoreMesh(core_axis_name="c", subcore_axis_name="s")

    @functools.partial(
        pl.kernel,
        mesh=mesh,
        out_type=jax.ShapeDtypeStruct((NC, N_PAD, D), jnp.float32),
        scratch_types=[
            pltpu.VMEM_SHARED((N_PAD, D), jnp.float32),
            pltpu.VMEM_SHARED((N_PAD, D), jnp.float32),
            pltpu.VMEM((CH,), jnp.int32),
            pltpu.VMEM((CH,), jnp.int32),
            pltpu.VMEM((CH, D), jnp.float32),
            pltpu.SemaphoreType.DMA,
        ],
        compiler_params=pltpu.CompilerParams(use_tc_tiling_on_sc=False),
    )
    def k(x_hbm, row_hbm, col_hbm, z_hbm, out_hbm,
          x_s, agg_s, row_v, col_v, rows_v, sem):
        c = lax.axis_index("c")
        s = lax.axis_index("s")
        wid = c * NS + s
        nbase = s * ROWS_PER_TILE
        pltpu.sync_copy(x_hbm.at[pl.ds(nbase, ROWS_PER_TILE)],
                        x_s.at[pl.ds(nbase, ROWS_PER_TILE)])
        pltpu.sync_copy(z_hbm.at[pl.ds(nbase, ROWS_PER_TILE)],
                        agg_s.at[pl.ds(nbase, ROWS_PER_TILE)])
        plsc.subcore_barrier()

        ebase = wid * E_PER_W

        def body(i, carry):
            base = ebase + i * CH
            pltpu.sync_copy(row_hbm.at[pl.ds(base, CH)], row_v)
            pltpu.sync_copy(col_hbm.at[pl.ds(base, CH)], col_v)
            pltpu.async_copy(x_s.at[row_v], rows_v, sem).wait()
            pltpu.sync_copy(rows_v, agg_s.at[col_v], add=True)
            return carry

        lax.fori_loop(0, ITERS, body, 0)
        plsc.subcore_barrier()
        pltpu.sync_copy(agg_s.at[pl.ds(nbase, ROWS_PER_TILE)],
                        out_hbm.at[c, pl.ds(nbase, ROWS_PER_TILE)])

    return k(x_pad, row, col, zeros)


def _mlp_body(x_ref, p_ref, w1x_ref, w1a_ref, b1_ref, w2_ref, b2_ref,
              w3_ref, b3_ref, out_ref):
    agg = p_ref[0] + p_ref[1]
    h = jnp.dot(x_ref[...], w1x_ref[...], preferred_element_type=jnp.float32)
    h += jnp.dot(agg, w1a_ref[...], preferred_element_type=jnp.float32)
    h = jax.nn.relu(h + b1_ref[...])
    h = jax.nn.relu(
        jnp.dot(h, w2_ref[...], preferred_element_type=jnp.float32)
        + b2_ref[...])
    out_ref[...] = (
        jnp.dot(h, w3_ref[...], preferred_element_type=jnp.float32)
        + b3_ref[...])


def _mlp(x, partials, W1, b1, W2, b2, W3, b3):
    R = 2000
    nblocks = N_NODES // R
    w1x = W1[:3]
    w1a = jnp.pad(W1[3:6], ((0, D - 3), (0, 0)))
    full = lambda i: (0, 0)
    return pl.pallas_call(
        _mlp_body,
        grid=(nblocks,),
        in_specs=[
            pl.BlockSpec((R, 3), lambda i: (i, 0)),
            pl.BlockSpec((2, R, D), lambda i: (0, i, 0)),
            pl.BlockSpec((3, 16), full),
            pl.BlockSpec((D, 16), full),
            pl.BlockSpec((1, 16), full),
            pl.BlockSpec((16, 16), full),
            pl.BlockSpec((1, 16), full),
            pl.BlockSpec((16, 3), full),
            pl.BlockSpec((1, 3), full),
        ],
        out_specs=pl.BlockSpec((R, 3), lambda i: (i, 0)),
        out_shape=jax.ShapeDtypeStruct((N_NODES, 3), jnp.float32),
    )(x, partials, w1x, w1a, b1.reshape(1, 16), W2, b2.reshape(1, 16),
      W3, b3.reshape(1, 3))


def kernel(x, edge_index, edge_attr, u, batch, W1, b1, W2, b2, W3, b3):
    pad_n = E_PAD - N_EDGES
    row = jnp.pad(edge_index[0].astype(jnp.int32), (0, pad_n),
                  constant_values=N_PAD - 1)
    col = jnp.pad(edge_index[1].astype(jnp.int32), (0, pad_n),
                  constant_values=N_PAD - 1)
    x_pad = jnp.pad(x, ((0, N_PAD - N_NODES), (0, D - 3)))
    zeros = jnp.zeros((N_PAD, D), jnp.float32)
    partials = _sc_agg(x_pad, row, col, zeros)
    return _mlp(x, partials, W1, b1, W2, b2, W3, b3)


# speedup vs baseline: 49.3439x; 1.0722x over previous
import functools

import jax
import jax.numpy as jnp
from jax import lax
from jax.experimental import pallas as pl
from jax.experimental.pallas import tpu as pltpu
from jax.experimental.pallas import tpu_sc as plsc

N_NODES = 100000
N_PAD = 100096
N_EDGES = 3200000
D = 8
NC = 2
NS = 16
NW = NC * NS
CH = 1408
ITERS = 72
E_PER_W = CH * ITERS
E_PAD = NW * E_PER_W
ROWS_PER_TILE = N_PAD // NS


def _sc_agg(x_pad, row, col, zeros):
    mesh = plsc.VectorSubcoreMesh(core_axis_name="c", subcore_axis_name="s")

    @functools.partial(
        pl.kernel,
        mesh=mesh,
        out_type=jax.ShapeDtypeStruct((NC, N_PAD, D), jnp.float32),
        scratch_types=[
            pltpu.VMEM_SHARED((N_PAD, D), jnp.float32),
            pltpu.VMEM_SHARED((N_PAD, D), jnp.float32),
            pltpu.VMEM((2, CH), jnp.int32),
            pltpu.VMEM((3, CH), jnp.int32),
            pltpu.VMEM((2, CH, D), jnp.float32),
            pltpu.SemaphoreType.DMA((2,)),
            pltpu.SemaphoreType.DMA,
            pltpu.SemaphoreType.DMA((2,)),
        ],
        compiler_params=pltpu.CompilerParams(use_tc_tiling_on_sc=False),
    )
    def k(x_hbm, row_hbm, col_hbm, z_hbm, out_hbm,
          x_s, agg_s, row_v, col_v, rows_v, sem_i, sem_g, sem_s):
        c = lax.axis_index("c")
        s = lax.axis_index("s")
        wid = c * NS + s
        nbase = s * ROWS_PER_TILE
        pltpu.sync_copy(x_hbm.at[pl.ds(nbase, ROWS_PER_TILE)],
                        x_s.at[pl.ds(nbase, ROWS_PER_TILE)])
        pltpu.sync_copy(z_hbm.at[pl.ds(nbase, ROWS_PER_TILE)],
                        agg_s.at[pl.ds(nbase, ROWS_PER_TILE)])
        plsc.subcore_barrier()

        ebase = wid * E_PER_W

        pltpu.async_copy(row_hbm.at[pl.ds(ebase, CH)], row_v.at[0],
                         sem_i.at[0])
        pltpu.async_copy(col_hbm.at[pl.ds(ebase, CH)], col_v.at[0],
                         sem_i.at[0])

        def body(i, carry):
            p = lax.rem(i, 2)
            q = 1 - p
            c3 = lax.rem(i, 3)
            c3n = lax.rem(i + 1, 3)
            base = ebase + i * CH

            @pl.when(i >= 2)
            def _drain():
                pltpu.make_async_copy(rows_v.at[p], agg_s.at[col_v.at[c3n]],
                                      sem_s.at[p]).wait()

            @pl.when(i + 1 < ITERS)
            def _prefetch():
                nb = ebase + (i + 1) * CH
                pltpu.async_copy(row_hbm.at[pl.ds(nb, CH)], row_v.at[q],
                                 sem_i.at[q])
                pltpu.async_copy(col_hbm.at[pl.ds(nb, CH)], col_v.at[c3n],
                                 sem_i.at[q])

            pltpu.make_async_copy(row_hbm.at[pl.ds(base, CH)], row_v.at[p],
                                  sem_i.at[p]).wait()
            pltpu.make_async_copy(col_hbm.at[pl.ds(base, CH)], col_v.at[c3],
                                  sem_i.at[p]).wait()

            pltpu.async_copy(x_s.at[row_v.at[p]], rows_v.at[p], sem_g).wait()
            pltpu.async_copy(rows_v.at[p], agg_s.at[col_v.at[c3]],
                             sem_s.at[p], add=True)
            return carry

        lax.fori_loop(0, ITERS, body, 0)
        pltpu.make_async_copy(rows_v.at[0], agg_s.at[col_v.at[1]],
                              sem_s.at[0]).wait()
        pltpu.make_async_copy(rows_v.at[1], agg_s.at[col_v.at[2]],
                              sem_s.at[1]).wait()
        plsc.subcore_barrier()
        pltpu.sync_copy(agg_s.at[pl.ds(nbase, ROWS_PER_TILE)],
                        out_hbm.at[c, pl.ds(nbase, ROWS_PER_TILE)])

    return k(x_pad, row, col, zeros)


def _mlp_body(x_ref, p_ref, w1x_ref, w1a_ref, b1_ref, w2_ref, b2_ref,
              w3_ref, b3_ref, out_ref):
    agg = p_ref[0] + p_ref[1]
    h = jnp.dot(x_ref[...], w1x_ref[...], preferred_element_type=jnp.float32)
    h += jnp.dot(agg, w1a_ref[...], preferred_element_type=jnp.float32)
    h = jax.nn.relu(h + b1_ref[...])
    h = jax.nn.relu(
        jnp.dot(h, w2_ref[...], preferred_element_type=jnp.float32)
        + b2_ref[...])
    out_ref[...] = (
        jnp.dot(h, w3_ref[...], preferred_element_type=jnp.float32)
        + b3_ref[...])


def _mlp(x, partials, W1, b1, W2, b2, W3, b3):
    R = 2000
    nblocks = N_NODES // R
    w1x = W1[:3]
    w1a = jnp.pad(W1[3:6], ((0, D - 3), (0, 0)))
    full = lambda i: (0, 0)
    return pl.pallas_call(
        _mlp_body,
        grid=(nblocks,),
        in_specs=[
            pl.BlockSpec((R, 3), lambda i: (i, 0)),
            pl.BlockSpec((2, R, D), lambda i: (0, i, 0)),
            pl.BlockSpec((3, 16), full),
            pl.BlockSpec((D, 16), full),
            pl.BlockSpec((1, 16), full),
            pl.BlockSpec((16, 16), full),
            pl.BlockSpec((1, 16), full),
            pl.BlockSpec((16, 3), full),
            pl.BlockSpec((1, 3), full),
        ],
        out_specs=pl.BlockSpec((R, 3), lambda i: (i, 0)),
        out_shape=jax.ShapeDtypeStruct((N_NODES, 3), jnp.float32),
    )(x, partials, w1x, w1a, b1.reshape(1, 16), W2, b2.reshape(1, 16),
      W3, b3.reshape(1, 3))


def kernel(x, edge_index, edge_attr, u, batch, W1, b1, W2, b2, W3, b3):
    pad_n = E_PAD - N_EDGES
    row = jnp.pad(edge_index[0].astype(jnp.int32), (0, pad_n),
                  constant_values=N_PAD - 1)
    col = jnp.pad(edge_index[1].astype(jnp.int32), (0, pad_n),
                  constant_values=N_PAD - 1)
    x_pad = jnp.pad(x, ((0, N_PAD - N_NODES), (0, D - 3)))
    zeros = jnp.zeros((N_PAD, D), jnp.float32)
    partials = _sc_agg(x_pad, row, col, zeros)
    return _mlp(x, partials, W1, b1, W2, b2, W3, b3)


# speedup vs baseline: 63.3267x; 1.2834x over previous
import functools

import jax
import jax.numpy as jnp
from jax import lax
from jax.experimental import pallas as pl
from jax.experimental.pallas import tpu as pltpu
from jax.experimental.pallas import tpu_sc as plsc

N_NODES = 100000
N_PAD = 100096
N_EDGES = 3200000
NC = 2
NS = 16
NW = NC * NS
CH = 2000
E_PER_W = N_EDGES // NW
ITERS = E_PER_W // CH
ROWS_PER_TILE = N_PAD // NS


def _sc_agg(x0, x1, x2, row, col, zeros):
    mesh = plsc.VectorSubcoreMesh(core_axis_name="c", subcore_axis_name="s")

    @functools.partial(
        pl.kernel,
        mesh=mesh,
        out_type=jax.ShapeDtypeStruct((NC, 3, N_PAD), jnp.float32),
        scratch_types=[
            pltpu.VMEM_SHARED((N_PAD,), jnp.float32),
            pltpu.VMEM_SHARED((N_PAD,), jnp.float32),
            pltpu.VMEM_SHARED((N_PAD,), jnp.float32),
            pltpu.VMEM_SHARED((N_PAD,), jnp.float32),
            pltpu.VMEM_SHARED((N_PAD,), jnp.float32),
            pltpu.VMEM_SHARED((N_PAD,), jnp.float32),
            pltpu.VMEM((2, CH), jnp.int32),
            pltpu.VMEM((3, CH), jnp.int32),
            pltpu.VMEM((2, 3, CH), jnp.float32),
            pltpu.SemaphoreType.DMA((2,)),
            pltpu.SemaphoreType.DMA,
            pltpu.SemaphoreType.DMA((2,)),
        ],
        compiler_params=pltpu.CompilerParams(use_tc_tiling_on_sc=False),
    )
    def k(x0_hbm, x1_hbm, x2_hbm, row_hbm, col_hbm, z_hbm, out_hbm,
          x0_s, x1_s, x2_s, a0_s, a1_s, a2_s,
          row_v, col_v, vv, sem_i, sem_g, sem_s):
        c = lax.axis_index("c")
        s = lax.axis_index("s")
        wid = c * NS + s
        nbase = s * ROWS_PER_TILE
        nsl = pl.ds(nbase, ROWS_PER_TILE)
        xs = (x0_s, x1_s, x2_s)
        ags = (a0_s, a1_s, a2_s)
        for xk_hbm, xk_s in zip((x0_hbm, x1_hbm, x2_hbm), xs):
            pltpu.sync_copy(xk_hbm.at[nsl], xk_s.at[nsl])
        for ak_s in ags:
            pltpu.sync_copy(z_hbm.at[nsl], ak_s.at[nsl])
        plsc.subcore_barrier()

        ebase = wid * E_PER_W

        pltpu.async_copy(row_hbm.at[pl.ds(ebase, CH)], row_v.at[0],
                         sem_i.at[0])
        pltpu.async_copy(col_hbm.at[pl.ds(ebase, CH)], col_v.at[0],
                         sem_i.at[0])

        def body(i, carry):
            p = lax.rem(i, 2)
            q = 1 - p
            c3 = lax.rem(i, 3)
            c3n = lax.rem(i + 1, 3)
            base = ebase + i * CH

            @pl.when(i >= 2)
            def _drain():
                for k, ak_s in enumerate(ags):
                    pltpu.make_async_copy(vv.at[p, k],
                                          ak_s.at[col_v.at[c3n]],
                                          sem_s.at[p]).wait()

            @pl.when(i + 1 < ITERS)
            def _prefetch():
                nb = base + CH
                pltpu.async_copy(row_hbm.at[pl.ds(nb, CH)], row_v.at[q],
                                 sem_i.at[q])
                pltpu.async_copy(col_hbm.at[pl.ds(nb, CH)], col_v.at[c3n],
                                 sem_i.at[q])

            pltpu.make_async_copy(row_hbm.at[pl.ds(base, CH)], row_v.at[p],
                                  sem_i.at[p]).wait()
            pltpu.make_async_copy(col_hbm.at[pl.ds(base, CH)], col_v.at[c3],
                                  sem_i.at[p]).wait()

            for k, xk_s in enumerate(xs):
                pltpu.async_copy(xk_s.at[row_v.at[p]], vv.at[p, k], sem_g)
            for k, xk_s in enumerate(xs):
                pltpu.make_async_copy(xk_s.at[row_v.at[p]], vv.at[p, k],
                                      sem_g).wait()
            for k, ak_s in enumerate(ags):
                pltpu.async_copy(vv.at[p, k], ak_s.at[col_v.at[c3]],
                                 sem_s.at[p], add=True)
            return carry

        lax.fori_loop(0, ITERS, body, 0)
        for k, ak_s in enumerate(ags):
            pltpu.make_async_copy(vv.at[0, k], ak_s.at[col_v.at[0]],
                                  sem_s.at[0]).wait()
        for k, ak_s in enumerate(ags):
            pltpu.make_async_copy(vv.at[1, k], ak_s.at[col_v.at[1]],
                                  sem_s.at[1]).wait()
        plsc.subcore_barrier()
        for k, ak_s in enumerate(ags):
            pltpu.sync_copy(ak_s.at[nsl], out_hbm.at[c, k, nsl])

    return k(x0, x1, x2, row, col, zeros)


def _mlp_body(x_ref, p_ref, w1x_ref, w1a_ref, b1_ref, w2_ref, b2_ref,
              w3_ref, b3_ref, out_ref):
    agg = p_ref[0] + p_ref[1]
    h = jnp.dot(x_ref[...], w1x_ref[...], preferred_element_type=jnp.float32)
    h += lax.dot_general(agg, w1a_ref[...], (((0,), (0,)), ((), ())),
                         preferred_element_type=jnp.float32)
    h = jax.nn.relu(h + b1_ref[...])
    h = jax.nn.relu(
        jnp.dot(h, w2_ref[...], preferred_element_type=jnp.float32)
        + b2_ref[...])
    out_ref[...] = (
        jnp.dot(h, w3_ref[...], preferred_element_type=jnp.float32)
        + b3_ref[...])


def _mlp(x, partials, W1, b1, W2, b2, W3, b3):
    R = 2048
    nblocks = -(-N_NODES // R)
    w1x = W1[:3]
    w1a = W1[3:6]
    full = lambda i: (0, 0)
    return pl.pallas_call(
        _mlp_body,
        grid=(nblocks,),
        in_specs=[
            pl.BlockSpec((R, 3), lambda i: (i, 0)),
            pl.BlockSpec((2, 3, R), lambda i: (0, 0, i)),
            pl.BlockSpec((3, 16), full),
            pl.BlockSpec((3, 16), full),
            pl.BlockSpec((1, 16), full),
            pl.BlockSpec((16, 16), full),
            pl.BlockSpec((1, 16), full),
            pl.BlockSpec((16, 3), full),
            pl.BlockSpec((1, 3), full),
        ],
        out_specs=pl.BlockSpec((R, 3), lambda i: (i, 0)),
        out_shape=jax.ShapeDtypeStruct((N_NODES, 3), jnp.float32),
    )(x, partials, w1x, w1a, b1.reshape(1, 16), W2, b2.reshape(1, 16),
      W3, b3.reshape(1, 3))


def kernel(x, edge_index, edge_attr, u, batch, W1, b1, W2, b2, W3, b3):
    row = edge_index[0].astype(jnp.int32)
    col = edge_index[1].astype(jnp.int32)
    pad = (0, N_PAD - N_NODES)
    x0 = jnp.pad(x[:, 0], pad)
    x1 = jnp.pad(x[:, 1], pad)
    x2 = jnp.pad(x[:, 2], pad)
    zeros = jnp.zeros((N_PAD,), jnp.float32)
    partials = _sc_agg(x0, x1, x2, row, col, zeros)
    return _mlp(x, partials, W1, b1, W2, b2, W3, b3)


# speedup vs baseline: 67.0965x; 1.0595x over previous
import functools

import jax
import jax.numpy as jnp
from jax import lax
from jax.experimental import pallas as pl
from jax.experimental.pallas import tpu as pltpu
from jax.experimental.pallas import tpu_sc as plsc

N_NODES = 100000
N_PAD = 100096
N_EDGES = 3200000
NC = 2
NS = 16
NW = NC * NS
CH = 4000
E_PER_W = N_EDGES // NW
ITERS = E_PER_W // CH
ROWS_PER_TILE = N_PAD // NS


def _sc_agg(x0, x1, x2, row, col, zeros):
    mesh = plsc.VectorSubcoreMesh(core_axis_name="c", subcore_axis_name="s")

    @functools.partial(
        pl.kernel,
        mesh=mesh,
        out_type=jax.ShapeDtypeStruct((NC, 3, N_PAD), jnp.float32),
        scratch_types=[
            pltpu.VMEM_SHARED((N_PAD,), jnp.float32),
            pltpu.VMEM_SHARED((N_PAD,), jnp.float32),
            pltpu.VMEM_SHARED((N_PAD,), jnp.float32),
            pltpu.VMEM_SHARED((N_PAD,), jnp.float32),
            pltpu.VMEM_SHARED((N_PAD,), jnp.float32),
            pltpu.VMEM_SHARED((N_PAD,), jnp.float32),
            pltpu.VMEM((2, CH), jnp.int32),
            pltpu.VMEM((3, CH), jnp.int32),
            pltpu.VMEM((2, 3, CH), jnp.float32),
            pltpu.SemaphoreType.DMA((2,)),
            pltpu.SemaphoreType.DMA,
            pltpu.SemaphoreType.DMA((2,)),
        ],
        compiler_params=pltpu.CompilerParams(use_tc_tiling_on_sc=False),
    )
    def k(x0_hbm, x1_hbm, x2_hbm, row_hbm, col_hbm, z_hbm, out_hbm,
          x0_s, x1_s, x2_s, a0_s, a1_s, a2_s,
          row_v, col_v, vv, sem_i, sem_g, sem_s):
        c = lax.axis_index("c")
        s = lax.axis_index("s")
        wid = c * NS + s
        nbase = s * ROWS_PER_TILE
        nsl = pl.ds(nbase, ROWS_PER_TILE)
        xs = (x0_s, x1_s, x2_s)
        ags = (a0_s, a1_s, a2_s)
        for xk_hbm, xk_s in zip((x0_hbm, x1_hbm, x2_hbm), xs):
            pltpu.sync_copy(xk_hbm.at[nsl], xk_s.at[nsl])
        for ak_s in ags:
            pltpu.sync_copy(z_hbm.at[nsl], ak_s.at[nsl])
        plsc.subcore_barrier()

        ebase = wid * E_PER_W

        pltpu.async_copy(row_hbm.at[pl.ds(ebase, CH)], row_v.at[0],
                         sem_i.at[0])
        pltpu.async_copy(col_hbm.at[pl.ds(ebase, CH)], col_v.at[0],
                         sem_i.at[0])

        def body(i, carry):
            p = lax.rem(i, 2)
            q = 1 - p
            c3 = lax.rem(i, 3)
            c3n = lax.rem(i + 1, 3)
            base = ebase + i * CH

            @pl.when(i >= 2)
            def _drain():
                for k, ak_s in enumerate(ags):
                    pltpu.make_async_copy(vv.at[p, k],
                                          ak_s.at[col_v.at[c3n]],
                                          sem_s.at[p]).wait()

            @pl.when(i + 1 < ITERS)
            def _prefetch():
                nb = base + CH
                pltpu.async_copy(row_hbm.at[pl.ds(nb, CH)], row_v.at[q],
                                 sem_i.at[q])
                pltpu.async_copy(col_hbm.at[pl.ds(nb, CH)], col_v.at[c3n],
                                 sem_i.at[q])

            pltpu.make_async_copy(row_hbm.at[pl.ds(base, CH)], row_v.at[p],
                                  sem_i.at[p]).wait()
            pltpu.make_async_copy(col_hbm.at[pl.ds(base, CH)], col_v.at[c3],
                                  sem_i.at[p]).wait()

            for k, xk_s in enumerate(xs):
                pltpu.async_copy(xk_s.at[row_v.at[p]], vv.at[p, k], sem_g)
            for k, xk_s in enumerate(xs):
                pltpu.make_async_copy(xk_s.at[row_v.at[p]], vv.at[p, k],
                                      sem_g).wait()
            for k, ak_s in enumerate(ags):
                pltpu.async_copy(vv.at[p, k], ak_s.at[col_v.at[c3]],
                                 sem_s.at[p], add=True)
            return carry

        lax.fori_loop(0, ITERS, body, 0)
        for k, ak_s in enumerate(ags):
            pltpu.make_async_copy(vv.at[1, k], ak_s.at[col_v.at[2]],
                                  sem_s.at[1]).wait()
        for k, ak_s in enumerate(ags):
            pltpu.make_async_copy(vv.at[0, k], ak_s.at[col_v.at[0]],
                                  sem_s.at[0]).wait()
        plsc.subcore_barrier()
        for k, ak_s in enumerate(ags):
            pltpu.sync_copy(ak_s.at[nsl], out_hbm.at[c, k, nsl])

    return k(x0, x1, x2, row, col, zeros)


def _mlp_body(x_ref, p_ref, w1x_ref, w1a_ref, b1_ref, w2_ref, b2_ref,
              w3_ref, b3_ref, out_ref):
    agg = p_ref[0] + p_ref[1]
    h = jnp.dot(x_ref[...], w1x_ref[...], preferred_element_type=jnp.float32)
    h += lax.dot_general(agg, w1a_ref[...], (((0,), (0,)), ((), ())),
                         preferred_element_type=jnp.float32)
    h = jax.nn.relu(h + b1_ref[...])
    h = jax.nn.relu(
        jnp.dot(h, w2_ref[...], preferred_element_type=jnp.float32)
        + b2_ref[...])
    out_ref[...] = (
        jnp.dot(h, w3_ref[...], preferred_element_type=jnp.float32)
        + b3_ref[...])


def _mlp(x, partials, W1, b1, W2, b2, W3, b3):
    R = 4096
    nblocks = -(-N_NODES // R)
    w1x = W1[:3]
    w1a = W1[3:6]
    full = lambda i: (0, 0)
    return pl.pallas_call(
        _mlp_body,
        grid=(nblocks,),
        in_specs=[
            pl.BlockSpec((R, 3), lambda i: (i, 0)),
            pl.BlockSpec((2, 3, R), lambda i: (0, 0, i)),
            pl.BlockSpec((3, 16), full),
            pl.BlockSpec((3, 16), full),
            pl.BlockSpec((1, 16), full),
            pl.BlockSpec((16, 16), full),
            pl.BlockSpec((1, 16), full),
            pl.BlockSpec((16, 3), full),
            pl.BlockSpec((1, 3), full),
        ],
        out_specs=pl.BlockSpec((R, 3), lambda i: (i, 0)),
        out_shape=jax.ShapeDtypeStruct((N_NODES, 3), jnp.float32),
    )(x, partials, w1x, w1a, b1.reshape(1, 16), W2, b2.reshape(1, 16),
      W3, b3.reshape(1, 3))


def kernel(x, edge_index, edge_attr, u, batch, W1, b1, W2, b2, W3, b3):
    row = edge_index[0].astype(jnp.int32)
    col = edge_index[1].astype(jnp.int32)
    pad = (0, N_PAD - N_NODES)
    x0 = jnp.pad(x[:, 0], pad)
    x1 = jnp.pad(x[:, 1], pad)
    x2 = jnp.pad(x[:, 2], pad)
    zeros = jnp.zeros((N_PAD,), jnp.float32)
    partials = _sc_agg(x0, x1, x2, row, col, zeros)
    return _mlp(x, partials, W1, b1, W2, b2, W3, b3)
